# baseline (speedup 1.0000x reference)
import functools
import math

import jax
import jax.numpy as jnp
import numpy as np
from jax import lax
from jax.experimental import pallas as pl
from jax.experimental.pallas import tpu as pltpu
from jax.experimental.pallas import tpu_sc as plsc

_D = 768
_S = 4096
_B = 4
_NW = 32
_PW = _S // _NW
_CP = 8
_NCH = _PW // _CP
_ROWS = _B * _CP
_NT = _D // 2
_NJ2 = _NT // 16
_NSLOT = 4
_NGRP = _NCH // _NSLOT
_SCALE = math.sqrt(float(_D))


def _pe_consts():
    num_timescales = _NT
    log_timescale = math.log(10000.0) / (float(num_timescales) - 1.0)
    div = np.exp(
        np.arange(num_timescales, dtype=np.float64) * -log_timescale)
    s0 = (np.arange(_NW, dtype=np.float64) * _PW)[:, None]
    base = np.concatenate(
        [np.sin(s0 * div[None, :]), np.cos(s0 * div[None, :])],
        axis=1).astype(np.float32)
    steps = np.concatenate([np.cos(div), np.sin(div)]).astype(np.float32)
    return base.reshape(-1), steps


_PE_BASE, _PE_STEPS = _pe_consts()


def _sc_body(x_hbm, table_hbm, base_hbm, steps_hbm, out_hbm, idx_v,
             pst, stp,
             e0, e1, e2, e3,
             g0, g1, g2, g3, o0, o1, o2, o3):
    emb = [e0, e1, e2, e3]
    gsem = [g0, g1, g2, g3]
    osem = [o0, o1, o2, o3]
    wid = lax.axis_index("s") * 2 + lax.axis_index("c")
    for b in range(_B):
        pltpu.sync_copy(x_hbm.at[b, pl.ds(wid * _PW, _PW)],
                        idx_v.at[pl.ds(b * _PW, _PW)])
    pltpu.sync_copy(base_hbm.at[pl.ds(wid * _D, _D)], pst)
    pltpu.sync_copy(steps_hbm.at[pl.ds(0, _D)], stp)

    def start_io(c, s):
        for b in range(_B):
            pltpu.async_copy(
                table_hbm.at[idx_v.at[pl.ds(b * _PW + c * _CP, _CP)]],
                emb[s].at[pl.ds(b * _CP, _CP)], gsem[s])

    def wait_in(s):
        for b in range(_B):
            pltpu.make_async_copy(
                table_hbm.at[idx_v.at[pl.ds(0, _CP)]],
                emb[s].at[pl.ds(b * _CP, _CP)], gsem[s]).wait()

    for s in range(_NSLOT):
        start_io(s, s)

    def group_body(g, carry):
        for s in range(_NSLOT):
            c = g * _NSLOT + s
            wait_in(s)

            def pos_body(p, carry2, s=s):
                for j in range(_NJ2):
                    col = j * 16
                    sj = pst[pl.ds(col, 16)]
                    cj = pst[pl.ds(_NT + col, 16)]
                    for b in range(_B):
                        r = b * _CP + p
                        emb[s][r, pl.ds(col, 16)] = (
                            emb[s][r, pl.ds(col, 16)] * _SCALE + sj)
                        emb[s][r, pl.ds(_NT + col, 16)] = (
                            emb[s][r, pl.ds(_NT + col, 16)] * _SCALE + cj)
                    cwj = stp[pl.ds(col, 16)]
                    swj = stp[pl.ds(_NT + col, 16)]
                    pst[pl.ds(col, 16)] = sj * cwj + cj * swj
                    pst[pl.ds(_NT + col, 16)] = cj * cwj - sj * swj
                return carry2

            lax.fori_loop(0, _CP, pos_body, 0)
            for b in range(_B):
                pltpu.async_copy(
                    emb[s].at[pl.ds(b * _CP, _CP)],
                    out_hbm.at[b, pl.ds(wid * _PW + c * _CP, _CP)],
                    osem[s])
        for s in range(_NSLOT):
            for b in range(_B):
                pltpu.make_async_copy(
                    emb[s].at[pl.ds(b * _CP, _CP)],
                    out_hbm.at[b, pl.ds(0, _CP)], osem[s]).wait()

            @pl.when(g < _NGRP - 1)
            def _refill(g=g, s=s):
                start_io((g + 1) * _NSLOT + s, s)
        return carry

    lax.fori_loop(0, _NGRP, group_body, 0)


_sc_call = pl.kernel(
    _sc_body,
    out_type=jax.ShapeDtypeStruct((_B, _S, _D), jnp.float32),
    mesh=plsc.VectorSubcoreMesh(core_axis_name="c", subcore_axis_name="s"),
    scratch_types=(
        [pltpu.VMEM((_B * _PW,), jnp.int32)]
        + [pltpu.VMEM((_D,), jnp.float32)] * 2
        + [pltpu.VMEM((_ROWS, _D), jnp.float32)] * _NSLOT
        + [pltpu.SemaphoreType.DMA] * (2 * _NSLOT)
    ),
)


def kernel(x, table, training):
    del training
    base = jnp.asarray(_PE_BASE)
    steps = jnp.asarray(_PE_STEPS)
    return _sc_call(x.astype(jnp.int32), table, base, steps)


# speedup vs baseline: 1.4052x; 1.4052x over previous
import functools
import math

import jax
import jax.numpy as jnp
import numpy as np
from jax import lax
from jax.experimental import pallas as pl
from jax.experimental.pallas import tpu as pltpu
from jax.experimental.pallas import tpu_sc as plsc

_D = 768
_S = 4096
_B = 4
_NW = 32
_PW = _S // _NW
_CP = 8
_NCH = _PW // _CP
_ROWS = _B * _CP
_NJ = _D // 16
_NT = _D // 2
_SA = 64
_SB = _S // _SA
_NSLOT = 4
_NGRP = _NCH // _NSLOT
_SCALE = math.sqrt(float(_D))


def _pe_factors():
    log_timescale = math.log(10000.0) / (float(_NT) - 1.0)
    div32 = np.exp(
        np.arange(_NT, dtype=np.float32) * np.float32(-log_timescale))
    w = div32.astype(np.float64)
    a = (np.arange(_SA, dtype=np.float64) * _SB)[:, None] * w[None, :]
    b = np.arange(_SB, dtype=np.float64)[:, None] * w[None, :]
    return (np.sin(a).astype(np.float32), np.cos(a).astype(np.float32),
            np.sin(b).astype(np.float32), np.cos(b).astype(np.float32))


_SIN_A, _COS_A, _SIN_B, _COS_B = _pe_factors()


def _sc_body(x_hbm, table_hbm, pe_hbm, out_hbm, idx_v,
             e0, e1, e2, e3, p0, p1, p2, p3,
             g0, g1, g2, g3, q0, q1, q2, q3, o0, o1, o2, o3):
    emb = [e0, e1, e2, e3]
    peb = [p0, p1, p2, p3]
    gsem = [g0, g1, g2, g3]
    psem = [q0, q1, q2, q3]
    osem = [o0, o1, o2, o3]
    wid = lax.axis_index("s") * 2 + lax.axis_index("c")
    for b in range(_B):
        pltpu.sync_copy(x_hbm.at[b, pl.ds(wid * _PW, _PW)],
                        idx_v.at[pl.ds(b * _PW, _PW)])

    def start_io(c, s):
        for b in range(_B):
            pltpu.async_copy(
                table_hbm.at[idx_v.at[pl.ds(b * _PW + c * _CP, _CP)]],
                emb[s].at[pl.ds(b * _CP, _CP)], gsem[s])
        pltpu.async_copy(
            pe_hbm.at[pl.ds(wid * _PW + c * _CP, _CP)], peb[s], psem[s])

    def wait_in(s):
        for b in range(_B):
            pltpu.make_async_copy(
                table_hbm.at[idx_v.at[pl.ds(0, _CP)]],
                emb[s].at[pl.ds(b * _CP, _CP)], gsem[s]).wait()
        pltpu.make_async_copy(
            pe_hbm.at[pl.ds(0, _CP)], peb[s], psem[s]).wait()

    for s in range(_NSLOT):
        start_io(s, s)

    def group_body(g, carry):
        for s in range(_NSLOT):
            c = g * _NSLOT + s
            wait_in(s)

            def pos_body(p, carry2, s=s):
                for j in range(_NJ):
                    col = j * 16
                    pv = peb[s][p, pl.ds(col, 16)]
                    for b in range(_B):
                        r = b * _CP + p
                        emb[s][r, pl.ds(col, 16)] = (
                            emb[s][r, pl.ds(col, 16)] * _SCALE + pv)
                return carry2

            lax.fori_loop(0, _CP, pos_body, 0)
            for b in range(_B):
                pltpu.async_copy(
                    emb[s].at[pl.ds(b * _CP, _CP)],
                    out_hbm.at[b, pl.ds(wid * _PW + c * _CP, _CP)],
                    osem[s])
        for s in range(_NSLOT):
            for b in range(_B):
                pltpu.make_async_copy(
                    emb[s].at[pl.ds(b * _CP, _CP)],
                    out_hbm.at[b, pl.ds(0, _CP)], osem[s]).wait()

            @pl.when(g < _NGRP - 1)
            def _refill(g=g, s=s):
                start_io((g + 1) * _NSLOT + s, s)
        return carry

    lax.fori_loop(0, _NGRP, group_body, 0)


_sc_call = pl.kernel(
    _sc_body,
    out_type=jax.ShapeDtypeStruct((_B, _S, _D), jnp.float32),
    mesh=plsc.VectorSubcoreMesh(core_axis_name="c", subcore_axis_name="s"),
    scratch_types=(
        [pltpu.VMEM((_B * _PW,), jnp.int32)]
        + [pltpu.VMEM((_ROWS, _D), jnp.float32)] * _NSLOT
        + [pltpu.VMEM((_CP, _D), jnp.float32)] * _NSLOT
        + [pltpu.SemaphoreType.DMA] * (3 * _NSLOT)
    ),
)


def kernel(x, table, training):
    del training
    gate = (x[0, 0] * 0 + 1).astype(jnp.float32)
    sa = jnp.asarray(_SIN_A)[:, None, :] * gate
    ca = jnp.asarray(_COS_A)[:, None, :]
    sb = jnp.asarray(_SIN_B)[None, :, :]
    cb = jnp.asarray(_COS_B)[None, :, :]
    pe = jnp.concatenate(
        [sa * cb + ca * sb, ca * cb - sa * sb], axis=-1).reshape(_S, _D)
    return _sc_call(x.astype(jnp.int32), table, pe)
